# baseline (speedup 1.0000x reference)
import functools

import jax
import jax.numpy as jnp
from jax import lax
from jax.experimental import pallas as pl
from jax.experimental.pallas import tpu as pltpu
from jax.experimental.pallas import tpu_sc as plsc

B = 16384
ENT = 1000000
REL = 500
S = 43
TD = 21
RD = 64
NC = 2
NS = 16
NW = NC * NS
EPW = B // NW
NCH = 4
CH = EPW // NCH
NG = CH // 16


def _feat_body(ts_ref, freq_ref, amps_ref, phas_ref, o_ref):
    ts = ts_ref[...].reshape(1, -1)
    om = (1.0 / freq_ref[...]).reshape(TD, 1)
    am = amps_ref[...].reshape(TD, 1)
    ph = phas_ref[...].reshape(TD, 1)
    f = am * jnp.sin(ts * om + ph)
    o_ref[...] = f * f


def _feat_sq_t(timestamps, freq, amps, phas):
    blk = 2048
    return pl.pallas_call(
        _feat_body,
        grid=(B // blk,),
        in_specs=[
            pl.BlockSpec((blk,), lambda i: (i,)),
            pl.BlockSpec((TD,), lambda i: (0,)),
            pl.BlockSpec((TD,), lambda i: (0,)),
            pl.BlockSpec((TD,), lambda i: (0,)),
        ],
        out_specs=pl.BlockSpec((TD, blk), lambda i: (0, i)),
        out_shape=jax.ShapeDtypeStruct((TD, B), jnp.float32),
    )(timestamps, freq.reshape(TD), amps.reshape(TD), phas.reshape(TD))


def _sc_body(heads, rels, tails, eh, et, rf, ri, fsqt, out,
             hv, rv, tv, hbidx, tbidx,
             hh, ht, th, tt, rfb, rib, fb, outv, sem):
    wid = lax.axis_index("s") * NC + lax.axis_index("c")
    base = wid * EPW

    cps = [
        pltpu.async_copy(heads.at[pl.ds(base, EPW)], hv, sem),
        pltpu.async_copy(rels.at[pl.ds(base, EPW)], rv, sem),
        pltpu.async_copy(tails.at[pl.ds(base, EPW)], tv, sem),
        pltpu.async_copy(rf, rfb, sem),
        pltpu.async_copy(ri, rib, sem),
    ]
    for j in range(TD):
        cps.append(pltpu.async_copy(fsqt.at[j, pl.ds(base, EPW)],
                                    fb.at[j], sem))
    for cp in cps:
        cp.wait()

    for c in range(NCH):
        cb = c * CH

        def build(g, carry):
            src = pl.ds(cb + g * 16, 16)
            hvec = hv[src]
            tvec = tv[src]
            for d in range(S):
                dst = pl.ds(d * CH + g * 16, 16)
                hbidx[dst] = hvec + d * ENT
                tbidx[dst] = tvec + d * ENT
            return carry

        lax.fori_loop(0, NG, build, 0)

        gs = [
            pltpu.async_copy(eh.at[hbidx], hh, sem),
            pltpu.async_copy(eh.at[tbidx], ht, sem),
            pltpu.async_copy(et.at[hbidx], th, sem),
            pltpu.async_copy(et.at[tbidx], tt, sem),
        ]
        for g in gs:
            g.wait()

        def group(g, carry):
            sl = pl.ds(g * 16, 16)
            asl = pl.ds(cb + g * 16, 16)
            rvec = rv[asl]
            acc = jnp.zeros((16,), jnp.float32)
            for d in range(S):
                dsl = pl.ds(d * CH + g * 16, 16)
                r1 = plsc.load_gather(rfb, [rvec + d * REL])
                r2 = plsc.load_gather(rib, [rvec + d * REL])
                acc = (acc + hh[dsl] * r1 * tt[dsl]
                       + ht[dsl] * r2 * th[dsl])
            for j in range(TD):
                r1t = plsc.load_gather(rfb, [rvec + (S + j) * REL])
                r2t = plsc.load_gather(rib, [rvec + (S + j) * REL])
                acc = acc + fb[j, asl] * (r1t + r2t)
            outv[asl] = acc * 0.5
            return carry

        lax.fori_loop(0, NG, group, 0)

    pltpu.sync_copy(outv, out.at[pl.ds(base, EPW)])


_SC_SCRATCH = [
    pltpu.VMEM((EPW,), jnp.int32),
    pltpu.VMEM((EPW,), jnp.int32),
    pltpu.VMEM((EPW,), jnp.int32),
    pltpu.VMEM((S * CH,), jnp.int32),
    pltpu.VMEM((S * CH,), jnp.int32),
    pltpu.VMEM((S * CH,), jnp.float32),
    pltpu.VMEM((S * CH,), jnp.float32),
    pltpu.VMEM((S * CH,), jnp.float32),
    pltpu.VMEM((S * CH,), jnp.float32),
    pltpu.VMEM((RD * REL,), jnp.float32),
    pltpu.VMEM((RD * REL,), jnp.float32),
    pltpu.VMEM((TD, EPW), jnp.float32),
    pltpu.VMEM((EPW,), jnp.float32),
    pltpu.SemaphoreType.DMA,
]


@functools.partial(
    pl.kernel,
    out_type=jax.ShapeDtypeStruct((B,), jnp.float32),
    mesh=plsc.VectorSubcoreMesh(core_axis_name="c", subcore_axis_name="s"),
    scratch_types=_SC_SCRATCH,
    compiler_params=pltpu.CompilerParams(needs_layout_passes=False,
                                         use_tc_tiling_on_sc=False),
)
def _sc_score(*args):
    _sc_body(*args)


def kernel(heads, rels, tails, timestamps, ent_embs_h, ent_embs_t,
           rel_embs_f, rel_embs_i, freq, amps, phas):
    fsqt = _feat_sq_t(timestamps, freq, amps, phas)
    eh = ent_embs_h.T.reshape(-1)
    et = ent_embs_t.T.reshape(-1)
    rf = rel_embs_f.T.reshape(-1)
    ri = rel_embs_i.T.reshape(-1)
    return _sc_score(heads, rels, tails, eh, et, rf, ri, fsqt)
